# initial kernel scaffold (unmeasured)
import jax
import jax.numpy as jnp
from jax import lax
from jax.experimental import pallas as pl
from jax.experimental.pallas import tpu as pltpu


def kernel(
    x,
):
    def body(*refs):
        pass

    out_shape = jax.ShapeDtypeStruct(..., jnp.float32)
    return pl.pallas_call(body, out_shape=out_shape)(...)


# baseline (device time: 14534 ns/iter reference)
import jax
import jax.numpy as jnp
from jax import lax
from jax.experimental import pallas as pl
from jax.experimental.pallas import tpu as pltpu

_STAGE_MASKS = (1, 3, 4)


def kernel(x):
    m, n = x.shape

    def body(x_ref, out_ref, send_buf, recv_buf, send_sems, recv_sems):
        my_pos = lax.axis_index("i")

        barrier_sem = pltpu.get_barrier_semaphore()
        for mask in _STAGE_MASKS:
            pl.semaphore_signal(
                barrier_sem, inc=1,
                device_id=(my_pos ^ mask,),
                device_id_type=pl.DeviceIdType.MESH,
            )
        pl.semaphore_wait(barrier_sem, len(_STAGE_MASKS))

        acc = x_ref[:, :]
        for s, mask in enumerate(_STAGE_MASKS):
            partner = my_pos ^ mask
            send_buf[s, :, :] = acc.astype(jnp.bfloat16)
            rdma = pltpu.make_async_remote_copy(
                src_ref=send_buf.at[s],
                dst_ref=recv_buf.at[s],
                send_sem=send_sems.at[s],
                recv_sem=recv_sems.at[s],
                device_id=(partner,),
                device_id_type=pl.DeviceIdType.MESH,
            )
            rdma.start()
            rdma.wait()
            acc = acc + recv_buf[s, :, :].astype(jnp.float32)

        out_ref[:, :] = acc

    return pl.pallas_call(
        body,
        out_shape=jax.ShapeDtypeStruct((m, n), jnp.float32),
        in_specs=[pl.BlockSpec(memory_space=pltpu.VMEM)],
        out_specs=pl.BlockSpec(memory_space=pltpu.VMEM),
        scratch_shapes=[
            pltpu.VMEM((3, m, n), jnp.bfloat16),
            pltpu.VMEM((3, m, n), jnp.bfloat16),
            pltpu.SemaphoreType.DMA((3,)),
            pltpu.SemaphoreType.DMA((3,)),
        ],
        compiler_params=pltpu.CompilerParams(collective_id=0),
    )(x)


# device time: 12057 ns/iter; 1.2054x vs baseline; 1.2054x over previous
import jax
import jax.numpy as jnp
from jax import lax
from jax.experimental import pallas as pl
from jax.experimental.pallas import tpu as pltpu

_DIM_MASKS = (1, 3, 4)
_CHUNKS = ((0, 96), (96, 80), (176, 80))


def kernel(x):
    m, n = x.shape

    def body(x_ref, out_ref, send_buf, recv_buf, send_sems, recv_sems):
        my_pos = lax.axis_index("i")

        barrier_sem = pltpu.get_barrier_semaphore()
        for mask in _DIM_MASKS:
            pl.semaphore_signal(
                barrier_sem, inc=1,
                device_id=(my_pos ^ mask,),
                device_id_type=pl.DeviceIdType.MESH,
            )
        pl.semaphore_wait(barrier_sem, len(_DIM_MASKS))

        acc = x_ref[:, :]
        for s in range(3):
            send_buf[s, :, :] = acc.astype(jnp.bfloat16)
            rdmas = []
            for c, (row0, nrows) in enumerate(_CHUNKS):
                partner = my_pos ^ _DIM_MASKS[(c + s) % 3]
                rdma = pltpu.make_async_remote_copy(
                    src_ref=send_buf.at[s, pl.ds(row0, nrows)],
                    dst_ref=recv_buf.at[s, pl.ds(row0, nrows)],
                    send_sem=send_sems.at[s, c],
                    recv_sem=recv_sems.at[s, c],
                    device_id=(partner,),
                    device_id_type=pl.DeviceIdType.MESH,
                )
                rdma.start()
                rdmas.append(rdma)
            for rdma in rdmas:
                rdma.wait()
            acc = acc + recv_buf[s, :, :].astype(jnp.float32)

        out_ref[:, :] = acc

    return pl.pallas_call(
        body,
        out_shape=jax.ShapeDtypeStruct((m, n), jnp.float32),
        in_specs=[pl.BlockSpec(memory_space=pltpu.VMEM)],
        out_specs=pl.BlockSpec(memory_space=pltpu.VMEM),
        scratch_shapes=[
            pltpu.VMEM((3, m, n), jnp.bfloat16),
            pltpu.VMEM((3, m, n), jnp.bfloat16),
            pltpu.SemaphoreType.DMA((3, 3)),
            pltpu.SemaphoreType.DMA((3, 3)),
        ],
        compiler_params=pltpu.CompilerParams(collective_id=0),
    )(x)


# device time: 12051 ns/iter; 1.2060x vs baseline; 1.0005x over previous
import jax
import jax.numpy as jnp
from jax import lax
from jax.experimental import pallas as pl
from jax.experimental.pallas import tpu as pltpu

_DIM_MASKS = (1, 3, 4)
_CHUNKS = ((0, 96), (96, 80), (176, 80))


def kernel(x):
    m, n = x.shape

    def body(x_ref, out_ref, send_buf, recv_buf, send_sems, recv_sems):
        my_pos = lax.axis_index("i")

        barrier_sem = pltpu.get_barrier_semaphore()
        for mask in _DIM_MASKS:
            pl.semaphore_signal(
                barrier_sem, inc=1,
                device_id=(my_pos ^ mask,),
                device_id_type=pl.DeviceIdType.MESH,
            )
        pl.semaphore_wait(barrier_sem, len(_DIM_MASKS))

        acc = x_ref[:, :]
        all_rdmas = []
        for s in range(3):
            send_buf[s, :, :] = acc.astype(jnp.bfloat16)
            rdmas = []
            for c, (row0, nrows) in enumerate(_CHUNKS):
                partner = my_pos ^ _DIM_MASKS[(c + s) % 3]
                rdma = pltpu.make_async_remote_copy(
                    src_ref=send_buf.at[s, pl.ds(row0, nrows)],
                    dst_ref=recv_buf.at[s, pl.ds(row0, nrows)],
                    send_sem=send_sems.at[s, c],
                    recv_sem=recv_sems.at[s, c],
                    device_id=(partner,),
                    device_id_type=pl.DeviceIdType.MESH,
                )
                rdma.start()
                rdmas.append(rdma)
                all_rdmas.append(rdma)
            for rdma in rdmas:
                rdma.wait_recv()
            acc = acc + recv_buf[s, :, :].astype(jnp.float32)

        out_ref[:, :] = acc
        for rdma in all_rdmas:
            rdma.wait_send()

    return pl.pallas_call(
        body,
        out_shape=jax.ShapeDtypeStruct((m, n), jnp.float32),
        in_specs=[pl.BlockSpec(memory_space=pltpu.VMEM)],
        out_specs=pl.BlockSpec(memory_space=pltpu.VMEM),
        scratch_shapes=[
            pltpu.VMEM((3, m, n), jnp.bfloat16),
            pltpu.VMEM((3, m, n), jnp.bfloat16),
            pltpu.SemaphoreType.DMA((3, 3)),
            pltpu.SemaphoreType.DMA((3, 3)),
        ],
        compiler_params=pltpu.CompilerParams(collective_id=0),
    )(x)
